# baseline (speedup 1.0000x reference)
import jax
import jax.numpy as jnp
from jax.experimental import pallas as pl
from jax.experimental.pallas import tpu as pltpu

_S = 4096
_R = 512
_G = _S // _R
_RG = _R // 8
_C = _S // 128


def _chamfer_body(xc_ref, yr_ref, out_ref, yb_ref, colmin_ref, rowacc_ref):
    i = pl.program_id(0)

    @pl.when(i == 0)
    def _init():
        for c in range(3):
            yb_ref[8 * c:8 * c + 8, :] = jnp.broadcast_to(
                yr_ref[c:c + 1, :], (8, _S))
        colmin_ref[...] = jnp.full((8, _S), jnp.inf, dtype=jnp.float32)
        rowacc_ref[...] = jnp.zeros((1, 1), dtype=jnp.float32)

    def row_group(r, acc):
        base = r * 8
        xb0 = jnp.broadcast_to(xc_ref[pl.ds(base, 8), 0:1], (8, 128))
        xb1 = jnp.broadcast_to(xc_ref[pl.ds(base, 8), 1:2], (8, 128))
        xb2 = jnp.broadcast_to(xc_ref[pl.ds(base, 8), 2:3], (8, 128))
        rmin = jnp.full((8, 128), jnp.inf, dtype=jnp.float32)
        for c in range(_C):
            sl = slice(c * 128, (c + 1) * 128)
            dx = xb0 - yb_ref[0:8, sl]
            dy = xb1 - yb_ref[8:16, sl]
            dz = xb2 - yb_ref[16:24, sl]
            d2 = dx * dx + dy * dy + dz * dz
            rmin = jnp.minimum(rmin, d2)
            colmin_ref[:, sl] = jnp.minimum(colmin_ref[:, sl], d2)
        row_d2 = jnp.min(rmin, axis=1)
        return acc + jnp.sum(jnp.sqrt(row_d2))

    s = jax.lax.fori_loop(0, _RG, row_group, jnp.float32(0.0))
    rowacc_ref[...] = rowacc_ref[...] + s

    @pl.when(i == _G - 1)
    def _fin():
        col_d2 = jnp.min(colmin_ref[...], axis=0, keepdims=True)
        loss2 = jnp.sum(jnp.sqrt(col_d2)) / _S
        out_ref[...] = rowacc_ref[...] / _S + loss2


def kernel(X, Y):
    Xc = X[0]
    Yr = jnp.transpose(Y[0], (1, 0))
    out = pl.pallas_call(
        _chamfer_body,
        grid=(_G,),
        in_specs=[
            pl.BlockSpec((_R, 3), lambda i: (i, 0)),
            pl.BlockSpec((3, _S), lambda i: (0, 0)),
        ],
        out_specs=pl.BlockSpec((1, 1), lambda i: (0, 0)),
        out_shape=jax.ShapeDtypeStruct((1, 1), jnp.float32),
        scratch_shapes=[
            pltpu.VMEM((24, _S), jnp.float32),
            pltpu.VMEM((8, _S), jnp.float32),
            pltpu.VMEM((1, 1), jnp.float32),
        ],
    )(Xc, Yr)
    return out[0, 0]


# speedup vs baseline: 4.8727x; 4.8727x over previous
import jax
import jax.numpy as jnp
from jax.experimental import pallas as pl
from jax.experimental.pallas import tpu as pltpu

_S = 4096
_RG = _S // 8
_C = _S // 128


def _chamfer_body(xc_ref, yr_ref, out_ref, xb_ref, yb_ref, colmin_ref,
                  rmin_ref):
    for c in range(3):
        yb_ref[8 * c:8 * c + 8, :] = jnp.broadcast_to(
            yr_ref[c:c + 1, :], (8, _S))
    colmin_ref[...] = jnp.full((8, _S), jnp.inf, dtype=jnp.float32)

    for c in range(3):
        xb_ref[c * _S:(c + 1) * _S, :] = jnp.broadcast_to(
            xc_ref[:, c:c + 1], (_S, 128))

    def row_group(r, _):
        base = r * 8
        xb0 = xb_ref[pl.ds(base, 8), :]
        xb1 = xb_ref[pl.ds(_S + base, 8), :]
        xb2 = xb_ref[pl.ds(2 * _S + base, 8), :]
        rmin = jnp.full((8, 128), jnp.inf, dtype=jnp.float32)
        for c in range(_C):
            sl = slice(c * 128, (c + 1) * 128)
            dx = xb0 - yb_ref[0:8, sl]
            dy = xb1 - yb_ref[8:16, sl]
            dz = xb2 - yb_ref[16:24, sl]
            d2 = dx * dx + dy * dy + dz * dz
            rmin = jnp.minimum(rmin, d2)
            colmin_ref[:, sl] = jnp.minimum(colmin_ref[:, sl], d2)
        rmin_ref[pl.ds(base, 8), :] = rmin
        return 0

    jax.lax.fori_loop(0, _RG, row_group, 0)

    row_d2 = jnp.min(rmin_ref[...], axis=1)
    loss1 = jnp.sum(jnp.sqrt(row_d2)) / _S
    col_d2 = jnp.min(colmin_ref[...], axis=0, keepdims=True)
    loss2 = jnp.sum(jnp.sqrt(col_d2)) / _S
    out_ref[...] = jnp.full((1, 1), loss1 + loss2, dtype=jnp.float32)


def kernel(X, Y):
    Xc = X[0]
    Yr = jnp.transpose(Y[0], (1, 0))
    out = pl.pallas_call(
        _chamfer_body,
        out_shape=jax.ShapeDtypeStruct((1, 1), jnp.float32),
        scratch_shapes=[
            pltpu.VMEM((3 * _S, 128), jnp.float32),
            pltpu.VMEM((24, _S), jnp.float32),
            pltpu.VMEM((8, _S), jnp.float32),
            pltpu.VMEM((_S, 128), jnp.float32),
        ],
    )(Xc, Yr)
    return out[0, 0]


# speedup vs baseline: 5.2317x; 1.0737x over previous
import jax
import jax.numpy as jnp
from jax import lax
from jax.experimental import pallas as pl
from jax.experimental.pallas import tpu as pltpu

_S = 4096
_RG = _S // 16
_C = _S // 128


def _chamfer_body(xc_ref, yr_ref, out_ref, xb_ref, yb_ref, colmin_ref,
                  rmin_ref):
    for c in range(3):
        yb_ref[16 * c:16 * c + 16, :] = jnp.broadcast_to(
            yr_ref[c:c + 1, :], (16, _S)).astype(jnp.bfloat16)
    colmin_ref[...] = jnp.full((16, _S), jnp.inf, dtype=jnp.bfloat16)

    for c in range(3):
        xb_ref[c * _S:(c + 1) * _S, :] = jnp.broadcast_to(
            xc_ref[:, c:c + 1], (_S, 128)).astype(jnp.bfloat16)

    def row_group(r, _):
        base = r * 16
        xb0 = xb_ref[pl.ds(base, 16), :]
        xb1 = xb_ref[pl.ds(_S + base, 16), :]
        xb2 = xb_ref[pl.ds(2 * _S + base, 16), :]
        rmin = jnp.full((16, 128), jnp.inf, dtype=jnp.bfloat16)
        for c in range(_C):
            sl = slice(c * 128, (c + 1) * 128)
            dx = xb0 - yb_ref[0:16, sl]
            dy = xb1 - yb_ref[16:32, sl]
            dz = xb2 - yb_ref[32:48, sl]
            d2 = dx * dx + dy * dy + dz * dz
            rmin = jnp.minimum(rmin, d2)
            colmin_ref[:, sl] = jnp.minimum(colmin_ref[:, sl], d2)
        rmin_ref[pl.ds(base, 16), :] = rmin
        return 0

    lax.fori_loop(0, _RG, row_group, 0)

    row_d2 = jnp.min(rmin_ref[...], axis=1).astype(jnp.float32)
    loss1 = jnp.sum(jnp.sqrt(row_d2)) / _S
    col_d2 = jnp.min(colmin_ref[...], axis=0,
                     keepdims=True).astype(jnp.float32)
    loss2 = jnp.sum(jnp.sqrt(col_d2)) / _S
    out_ref[...] = jnp.full((1, 1), loss1 + loss2, dtype=jnp.float32)


def kernel(X, Y):
    Xc = X[0]
    Yr = jnp.transpose(Y[0], (1, 0))
    out = pl.pallas_call(
        _chamfer_body,
        out_shape=jax.ShapeDtypeStruct((1, 1), jnp.float32),
        scratch_shapes=[
            pltpu.VMEM((3 * _S, 128), jnp.bfloat16),
            pltpu.VMEM((48, _S), jnp.bfloat16),
            pltpu.VMEM((16, _S), jnp.bfloat16),
            pltpu.VMEM((_S, 128), jnp.bfloat16),
        ],
    )(Xc, Yr)
    return out[0, 0]
